# baseline (speedup 1.0000x reference)
import functools

import jax
import jax.numpy as jnp
from jax import lax
from jax.experimental import pallas as pl
from jax.experimental.pallas import tpu as pltpu
from jax.experimental.pallas import tpu_sc as plsc

N_OBS = 50000
N_GRAPH = 512
GF = 6
OPD = 256
B = 1024


_OC = 2000
_NOC = N_OBS // _OC


def _mxu_dot2(p0, p1):
    hi = jnp.maximum(p0, p1)
    lo = jnp.minimum(p0, p1)
    bits = lax.bitcast_convert_type(hi, jnp.int32)
    ebc = jnp.maximum((bits >> 23) & 0xFF, 28)
    grid = lax.bitcast_convert_type((ebc - 27) << 23, jnp.float32)
    ginv = lax.bitcast_convert_type((281 - ebc) << 23, jnp.float32)
    lo_t = jnp.floor(lo * ginv) * grid
    return jnp.where(lo == 0.0, hi, hi + lo_t)


def _argmin_body(pos_ref, obs_ref, idx_ref, minv_ref, mini_ref):
    k = pl.program_id(0)
    pos_t = jnp.transpose(pos_ref[...], (1, 0))
    px = pos_t[0:1, :]
    py = pos_t[1:2, :]
    q2 = px * px + py * py
    bpx = px.astype(jnp.bfloat16).astype(jnp.float32)
    bpy = py.astype(jnp.bfloat16).astype(jnp.float32)

    ox = obs_ref[:, 0:1]
    oy = obs_ref[:, 1:2]
    o2 = ox * ox + oy * oy
    box = ox.astype(jnp.bfloat16).astype(jnp.float32)
    boy = oy.astype(jnp.bfloat16).astype(jnp.float32)
    dot = _mxu_dot2(box * bpx, boy * bpy)
    d = (q2 + o2) - 2.0 * dot
    m = jnp.min(d, axis=0, keepdims=True)
    iota = lax.broadcasted_iota(jnp.int32, (_OC, B), 0)
    la = jnp.min(jnp.where(d == m, iota, _OC), axis=0, keepdims=True)
    gi = k * _OC + la

    @pl.when(k == 0)
    def _init():
        minv_ref[...] = jnp.full((1, B), jnp.inf, jnp.float32)
        mini_ref[...] = jnp.zeros((1, B), jnp.int32)

    better = m < minv_ref[...]
    minv_ref[...] = jnp.where(better, m, minv_ref[...])
    mini_ref[...] = jnp.where(better, gi, mini_ref[...])

    @pl.when(k == _NOC - 1)
    def _fin():
        idx_ref[...] = mini_ref[...]


def _argmin_call(positions, obs_pos):
    return pl.pallas_call(
        _argmin_body,
        grid=(_NOC,),
        in_specs=[
            pl.BlockSpec((B, 2), lambda k: (0, 0)),
            pl.BlockSpec((_OC, 2), lambda k: (k, 0)),
        ],
        out_specs=pl.BlockSpec((1, B), lambda k: (0, 0)),
        out_shape=jax.ShapeDtypeStruct((1, B), jnp.int32),
        scratch_shapes=[
            pltpu.VMEM((1, B), jnp.float32),
            pltpu.VMEM((1, B), jnp.int32),
        ],
    )(positions, obs_pos)


_NC = 2
_NS = 16
_NW = _NC * _NS
_B_PER_W = B // _NW


@functools.lru_cache(maxsize=None)
def _make_gather_sc():
    @functools.partial(
        pl.kernel,
        mesh=plsc.VectorSubcoreMesh(core_axis_name="c", subcore_axis_name="s"),
        out_type=jax.ShapeDtypeStruct((B, N_GRAPH), jnp.float32),
        scratch_types=[
            pltpu.VMEM((_B_PER_W,), jnp.int32),
            pltpu.VMEM((_B_PER_W, N_GRAPH), jnp.float32),
            pltpu.SemaphoreType.DMA,
        ],
    )
    def _gather_sc(idx_hbm, table_hbm, out_hbm, idx_v, rows_v, sem):
        wid = lax.axis_index("s") * _NC + lax.axis_index("c")
        base = wid * _B_PER_W
        pltpu.sync_copy(idx_hbm.at[pl.ds(base, _B_PER_W)], idx_v)
        pltpu.async_copy(table_hbm.at[idx_v], rows_v, sem).wait()
        pltpu.sync_copy(rows_v, out_hbm.at[pl.ds(base, _B_PER_W)])

    return _gather_sc


_BT = 256
_CT = 8192


def _expand_body(g_ref, a_ref, s_ref, out_ref):
    inter = jnp.dot(g_ref[...], a_ref[...],
                    preferred_element_type=jnp.float32)
    out_ref[...] = jnp.dot(inter, s_ref[...],
                           preferred_element_type=jnp.float32)


def _expand_call(gathered, alpha, s_flat):
    nb = B // _BT
    nc = (OPD * OPD) // _CT
    return pl.pallas_call(
        _expand_body,
        grid=(nb, nc),
        in_specs=[
            pl.BlockSpec((_BT, N_GRAPH), lambda i, j: (i, 0)),
            pl.BlockSpec((N_GRAPH, GF), lambda i, j: (0, 0)),
            pl.BlockSpec((GF, _CT), lambda i, j: (0, j)),
        ],
        out_specs=pl.BlockSpec((_BT, _CT), lambda i, j: (i, j)),
        out_shape=jax.ShapeDtypeStruct((B, OPD * OPD), jnp.float32),
    )(gathered, alpha, s_flat)


def kernel(positions, obs_pos, graph_dic, S_graph, alpha_graph):
    idx = _argmin_call(positions, obs_pos).reshape(B)
    gathered = _make_gather_sc()(idx, graph_dic)
    s_flat = S_graph.reshape(GF, OPD * OPD)
    out = _expand_call(gathered, alpha_graph, s_flat).reshape(B, OPD, OPD)
    return (out, alpha_graph)


# speedup vs baseline: 1.1759x; 1.1759x over previous
import functools

import jax
import jax.numpy as jnp
from jax import lax
from jax.experimental import pallas as pl
from jax.experimental.pallas import tpu as pltpu
from jax.experimental.pallas import tpu_sc as plsc

N_OBS = 50000
N_GRAPH = 512
GF = 6
OPD = 256
B = 1024


_OC = 2000
_NOC = N_OBS // _OC


def _mxu_dot2(p0, p1):
    hi = jnp.maximum(p0, p1)
    lo = jnp.minimum(p0, p1)
    bits = lax.bitcast_convert_type(hi, jnp.int32)
    ebc = jnp.maximum((bits >> 23) & 0xFF, 28)
    grid = lax.bitcast_convert_type((ebc - 27) << 23, jnp.float32)
    ginv = lax.bitcast_convert_type((281 - ebc) << 23, jnp.float32)
    lo_t = jnp.floor(lo * ginv) * grid
    return jnp.where(lo == 0.0, hi, hi + lo_t)


def _argmin_body(pos_ref, obs_ref, idx_ref, minv_ref, mini_ref):
    k = pl.program_id(0)
    pos_t = jnp.transpose(pos_ref[...], (1, 0))
    px = pos_t[0:1, :]
    py = pos_t[1:2, :]
    q2 = px * px + py * py
    bpx = px.astype(jnp.bfloat16).astype(jnp.float32)
    bpy = py.astype(jnp.bfloat16).astype(jnp.float32)

    ox = obs_ref[:, 0:1]
    oy = obs_ref[:, 1:2]
    o2 = ox * ox + oy * oy
    box = ox.astype(jnp.bfloat16).astype(jnp.float32)
    boy = oy.astype(jnp.bfloat16).astype(jnp.float32)
    dot = _mxu_dot2(box * bpx, boy * bpy)
    d = (q2 + o2) - 2.0 * dot
    m = jnp.min(d, axis=0, keepdims=True)
    iota = lax.broadcasted_iota(jnp.int32, (_OC, B), 0)
    la = jnp.min(jnp.where(d == m, iota, _OC), axis=0, keepdims=True)
    gi = k * _OC + la

    @pl.when(k == 0)
    def _init():
        minv_ref[...] = jnp.full((1, B), jnp.inf, jnp.float32)
        mini_ref[...] = jnp.zeros((1, B), jnp.int32)

    better = m < minv_ref[...]
    minv_ref[...] = jnp.where(better, m, minv_ref[...])
    mini_ref[...] = jnp.where(better, gi, mini_ref[...])

    @pl.when(k == _NOC - 1)
    def _fin():
        idx_ref[...] = mini_ref[...]


def _argmin_call(positions, obs_pos):
    return pl.pallas_call(
        _argmin_body,
        grid=(_NOC,),
        in_specs=[
            pl.BlockSpec((B, 2), lambda k: (0, 0)),
            pl.BlockSpec((_OC, 2), lambda k: (k, 0)),
        ],
        out_specs=pl.BlockSpec((1, B), lambda k: (0, 0)),
        out_shape=jax.ShapeDtypeStruct((1, B), jnp.int32),
        scratch_shapes=[
            pltpu.VMEM((1, B), jnp.float32),
            pltpu.VMEM((1, B), jnp.int32),
        ],
    )(positions, obs_pos)


_NC = 2
_NS = 16
_NW = _NC * _NS
_B_PER_W = B // _NW


@functools.lru_cache(maxsize=None)
def _make_gather_sc():
    @functools.partial(
        pl.kernel,
        mesh=plsc.VectorSubcoreMesh(core_axis_name="c", subcore_axis_name="s"),
        out_type=jax.ShapeDtypeStruct((B, N_GRAPH), jnp.float32),
        scratch_types=[
            pltpu.VMEM((_B_PER_W,), jnp.int32),
            pltpu.VMEM((_B_PER_W, N_GRAPH), jnp.float32),
            pltpu.SemaphoreType.DMA,
        ],
    )
    def _gather_sc(idx_hbm, table_hbm, out_hbm, idx_v, rows_v, sem):
        wid = lax.axis_index("s") * _NC + lax.axis_index("c")
        base = wid * _B_PER_W
        pltpu.sync_copy(idx_hbm.at[pl.ds(base, _B_PER_W)], idx_v)
        pltpu.async_copy(table_hbm.at[idx_v], rows_v, sem).wait()
        pltpu.sync_copy(rows_v, out_hbm.at[pl.ds(base, _B_PER_W)])

    return _gather_sc


_BB = 16


def _expand_body(g_ref, a_ref, s_ref, out_ref):
    inter = jnp.dot(g_ref[...], a_ref[...],
                    preferred_element_type=jnp.float32)
    for lb in range(_BB):
        acc = inter[lb:lb + 1, 0:1] * s_ref[0:OPD, :]
        for f in range(1, GF):
            acc = acc + inter[lb:lb + 1, f:f + 1] * s_ref[f * OPD:(f + 1) * OPD, :]
        out_ref[lb * OPD:(lb + 1) * OPD, :] = acc


def _expand_call(gathered, alpha, s2):
    return pl.pallas_call(
        _expand_body,
        grid=(B // _BB,),
        in_specs=[
            pl.BlockSpec((_BB, N_GRAPH), lambda i: (i, 0)),
            pl.BlockSpec((N_GRAPH, GF), lambda i: (0, 0)),
            pl.BlockSpec((GF * OPD, OPD), lambda i: (0, 0)),
        ],
        out_specs=pl.BlockSpec((_BB * OPD, OPD), lambda i: (i, 0)),
        out_shape=jax.ShapeDtypeStruct((B * OPD, OPD), jnp.float32),
    )(gathered, alpha, s2)


def kernel(positions, obs_pos, graph_dic, S_graph, alpha_graph):
    idx = _argmin_call(positions, obs_pos).reshape(B)
    gathered = _make_gather_sc()(idx, graph_dic)
    s2 = S_graph.reshape(GF * OPD, OPD)
    out = _expand_call(gathered, alpha_graph, s2).reshape(B, OPD, OPD)
    return (out, alpha_graph)


# speedup vs baseline: 1.6255x; 1.3824x over previous
import functools

import jax
import jax.numpy as jnp
from jax import lax
from jax.experimental import pallas as pl
from jax.experimental.pallas import tpu as pltpu
from jax.experimental.pallas import tpu_sc as plsc

N_OBS = 50000
N_GRAPH = 512
GF = 6
OPD = 256
B = 1024


_OC = 2000
_NOC = N_OBS // _OC


def _argmin_body(pos_ref, obs_ref, idx_ref, minv_ref, mini_ref):
    k = pl.program_id(0)
    pos_t = jnp.transpose(pos_ref[...], (1, 0))
    px = pos_t[0:1, :]
    py = pos_t[1:2, :]
    q2 = px * px + py * py

    ox = obs_ref[:, 0:1]
    oy = obs_ref[:, 1:2]
    o2 = ox * ox + oy * oy
    dot = lax.dot_general(obs_ref[...], pos_t,
                          dimension_numbers=(((1,), (0,)), ((), ())),
                          preferred_element_type=jnp.float32)
    d = (q2 + o2) - 2.0 * dot
    m = jnp.min(d, axis=0, keepdims=True)
    iota = lax.broadcasted_iota(jnp.int32, (_OC, B), 0)
    la = jnp.min(jnp.where(d == m, iota, _OC), axis=0, keepdims=True)
    gi = k * _OC + la

    @pl.when(k == 0)
    def _init():
        minv_ref[...] = jnp.full((1, B), jnp.inf, jnp.float32)
        mini_ref[...] = jnp.zeros((1, B), jnp.int32)

    better = m < minv_ref[...]
    minv_ref[...] = jnp.where(better, m, minv_ref[...])
    mini_ref[...] = jnp.where(better, gi, mini_ref[...])

    @pl.when(k == _NOC - 1)
    def _fin():
        idx_ref[...] = mini_ref[...]


def _argmin_call(positions, obs_pos):
    return pl.pallas_call(
        _argmin_body,
        grid=(_NOC,),
        in_specs=[
            pl.BlockSpec((B, 2), lambda k: (0, 0)),
            pl.BlockSpec((_OC, 2), lambda k: (k, 0)),
        ],
        out_specs=pl.BlockSpec((1, B), lambda k: (0, 0)),
        out_shape=jax.ShapeDtypeStruct((1, B), jnp.int32),
        scratch_shapes=[
            pltpu.VMEM((1, B), jnp.float32),
            pltpu.VMEM((1, B), jnp.int32),
        ],
    )(positions, obs_pos)


_NC = 2
_NS = 16
_NW = _NC * _NS
_B_PER_W = B // _NW


@functools.lru_cache(maxsize=None)
def _make_gather_sc():
    @functools.partial(
        pl.kernel,
        mesh=plsc.VectorSubcoreMesh(core_axis_name="c", subcore_axis_name="s"),
        out_type=jax.ShapeDtypeStruct((B, N_GRAPH), jnp.float32),
        scratch_types=[
            pltpu.VMEM((_B_PER_W,), jnp.int32),
            pltpu.VMEM((_B_PER_W, N_GRAPH), jnp.float32),
            pltpu.SemaphoreType.DMA,
        ],
    )
    def _gather_sc(idx_hbm, table_hbm, out_hbm, idx_v, rows_v, sem):
        wid = lax.axis_index("s") * _NC + lax.axis_index("c")
        base = wid * _B_PER_W
        pltpu.sync_copy(idx_hbm.at[pl.ds(base, _B_PER_W)], idx_v)
        pltpu.async_copy(table_hbm.at[idx_v], rows_v, sem).wait()
        pltpu.sync_copy(rows_v, out_hbm.at[pl.ds(base, _B_PER_W)])

    return _gather_sc


_BB = 16


def _expand_body(g_ref, a_ref, s_ref, out_ref):
    inter = jnp.dot(g_ref[...], a_ref[...],
                    preferred_element_type=jnp.float32)
    for lb in range(_BB):
        acc = inter[lb:lb + 1, 0:1] * s_ref[0:OPD, :]
        for f in range(1, GF):
            acc = acc + inter[lb:lb + 1, f:f + 1] * s_ref[f * OPD:(f + 1) * OPD, :]
        out_ref[lb * OPD:(lb + 1) * OPD, :] = acc


def _expand_call(gathered, alpha, s2):
    return pl.pallas_call(
        _expand_body,
        grid=(B // _BB,),
        in_specs=[
            pl.BlockSpec((_BB, N_GRAPH), lambda i: (i, 0)),
            pl.BlockSpec((N_GRAPH, GF), lambda i: (0, 0)),
            pl.BlockSpec((GF * OPD, OPD), lambda i: (0, 0)),
        ],
        out_specs=pl.BlockSpec((_BB * OPD, OPD), lambda i: (i, 0)),
        out_shape=jax.ShapeDtypeStruct((B * OPD, OPD), jnp.float32),
    )(gathered, alpha, s2)


def kernel(positions, obs_pos, graph_dic, S_graph, alpha_graph):
    idx = _argmin_call(positions, obs_pos).reshape(B)
    gathered = _make_gather_sc()(idx, graph_dic)
    s2 = S_graph.reshape(GF * OPD, OPD)
    out = _expand_call(gathered, alpha_graph, s2).reshape(B, OPD, OPD)
    return (out, alpha_graph)


# speedup vs baseline: 2.1022x; 1.2932x over previous
import functools

import jax
import jax.numpy as jnp
from jax import lax
from jax.experimental import pallas as pl
from jax.experimental.pallas import tpu as pltpu
from jax.experimental.pallas import tpu_sc as plsc

N_OBS = 50000
N_GRAPH = 512
GF = 6
OPD = 256
B = 1024


_OC = 2000
_NOC = N_OBS // _OC


def _argmin_body(pos_ref, obs_ref, idx_ref, minv_ref, mini_ref):
    k = pl.program_id(0)
    pos_t = jnp.transpose(pos_ref[...], (1, 0))
    px = pos_t[0:1, :]
    py = pos_t[1:2, :]
    q2 = px * px + py * py

    ox = obs_ref[:, 0:1]
    oy = obs_ref[:, 1:2]
    o2 = ox * ox + oy * oy
    dot = lax.dot_general(obs_ref[...], pos_t,
                          dimension_numbers=(((1,), (0,)), ((), ())),
                          preferred_element_type=jnp.float32)
    d = (q2 + o2) - 2.0 * dot
    m = jnp.min(d, axis=0, keepdims=True)
    iota = lax.broadcasted_iota(jnp.int32, (_OC, B), 0)
    la = jnp.min(jnp.where(d == m, iota, _OC), axis=0, keepdims=True)
    gi = k * _OC + la

    @pl.when(k == 0)
    def _init():
        minv_ref[...] = jnp.full((1, B), jnp.inf, jnp.float32)
        mini_ref[...] = jnp.zeros((1, B), jnp.int32)

    better = m < minv_ref[...]
    minv_ref[...] = jnp.where(better, m, minv_ref[...])
    mini_ref[...] = jnp.where(better, gi, mini_ref[...])

    @pl.when(k == _NOC - 1)
    def _fin():
        idx_ref[...] = mini_ref[...]


def _argmin_call(positions, obs_pos):
    return pl.pallas_call(
        _argmin_body,
        grid=(_NOC,),
        in_specs=[
            pl.BlockSpec((B, 2), lambda k: (0, 0)),
            pl.BlockSpec((_OC, 2), lambda k: (k, 0)),
        ],
        out_specs=pl.BlockSpec((1, B), lambda k: (0, 0)),
        out_shape=jax.ShapeDtypeStruct((1, B), jnp.int32),
        scratch_shapes=[
            pltpu.VMEM((1, B), jnp.float32),
            pltpu.VMEM((1, B), jnp.int32),
        ],
    )(positions, obs_pos)


_NC = 2
_NS = 16
_NW = _NC * _NS
_B_PER_W = B // _NW


@functools.lru_cache(maxsize=None)
def _make_gather_sc():
    @functools.partial(
        pl.kernel,
        mesh=plsc.VectorSubcoreMesh(core_axis_name="c", subcore_axis_name="s"),
        out_type=jax.ShapeDtypeStruct((B, N_GRAPH), jnp.float32),
        scratch_types=[
            pltpu.VMEM((_B_PER_W,), jnp.int32),
            pltpu.VMEM((_B_PER_W, N_GRAPH), jnp.float32),
            pltpu.SemaphoreType.DMA,
        ],
    )
    def _gather_sc(idx_hbm, table_hbm, out_hbm, idx_v, rows_v, sem):
        wid = lax.axis_index("s") * _NC + lax.axis_index("c")
        base = wid * _B_PER_W
        pltpu.sync_copy(idx_hbm.at[pl.ds(base, _B_PER_W)], idx_v)
        pltpu.async_copy(table_hbm.at[idx_v], rows_v, sem).wait()
        pltpu.sync_copy(rows_v, out_hbm.at[pl.ds(base, _B_PER_W)])

    return _gather_sc


_BB = 16


def _expand_body(g_ref, a_ref, s_ref, out_ref):
    inter = jnp.dot(g_ref[...], a_ref[...],
                    preferred_element_type=jnp.float32)
    for lbg in range(_BB // 4):
        cs = [[inter[lbg * 4 + q:lbg * 4 + q + 1, f:f + 1] for f in range(GF)]
              for q in range(4)]
        for it in range(OPD // 8):
            r0 = it * 8
            s_tiles = [s_ref[f * OPD + r0:f * OPD + r0 + 8, :] for f in range(GF)]
            for q in range(4):
                lb = lbg * 4 + q
                acc = cs[q][0] * s_tiles[0]
                for f in range(1, GF):
                    acc = acc + cs[q][f] * s_tiles[f]
                out_ref[lb * OPD + r0:lb * OPD + r0 + 8, :] = acc


def _expand_call(gathered, alpha, s2):
    return pl.pallas_call(
        _expand_body,
        grid=(B // _BB,),
        in_specs=[
            pl.BlockSpec((_BB, N_GRAPH), lambda i: (i, 0)),
            pl.BlockSpec((N_GRAPH, GF), lambda i: (0, 0)),
            pl.BlockSpec((GF * OPD, OPD), lambda i: (0, 0)),
        ],
        out_specs=pl.BlockSpec((_BB * OPD, OPD), lambda i: (i, 0)),
        out_shape=jax.ShapeDtypeStruct((B * OPD, OPD), jnp.float32),
    )(gathered, alpha, s2)


def kernel(positions, obs_pos, graph_dic, S_graph, alpha_graph):
    idx = _argmin_call(positions, obs_pos).reshape(B)
    gathered = _make_gather_sc()(idx, graph_dic)
    s2 = S_graph.reshape(GF * OPD, OPD)
    out = _expand_call(gathered, alpha_graph, s2).reshape(B, OPD, OPD)
    return (out, alpha_graph)


# speedup vs baseline: 2.1562x; 1.0257x over previous
import functools

import jax
import jax.numpy as jnp
from jax import lax
from jax.experimental import pallas as pl
from jax.experimental.pallas import tpu as pltpu
from jax.experimental.pallas import tpu_sc as plsc

N_OBS = 50000
N_GRAPH = 512
GF = 6
OPD = 256
B = 1024


_OC = 2000
_NOC = N_OBS // _OC


def _argmin_body(pos_ref, obs_ref, idx_ref, minv_ref, mini_ref):
    k = pl.program_id(0)
    pos_t = jnp.transpose(pos_ref[...], (1, 0))
    px = pos_t[0:1, :]
    py = pos_t[1:2, :]
    q2 = px * px + py * py

    ox = obs_ref[:, 0:1]
    oy = obs_ref[:, 1:2]
    o2 = ox * ox + oy * oy
    dot = lax.dot_general(obs_ref[...], pos_t,
                          dimension_numbers=(((1,), (0,)), ((), ())),
                          preferred_element_type=jnp.float32)
    d = (q2 + o2) - 2.0 * dot
    m = jnp.min(d, axis=0, keepdims=True)
    iota = lax.broadcasted_iota(jnp.int32, (_OC, 1), 0).astype(jnp.float32)
    la = jnp.min(jnp.where(d == m, iota, jnp.float32(_OC)),
                 axis=0, keepdims=True)
    gi = k * _OC + la.astype(jnp.int32)

    @pl.when(k == 0)
    def _init():
        minv_ref[...] = jnp.full((1, B), jnp.inf, jnp.float32)
        mini_ref[...] = jnp.zeros((1, B), jnp.int32)

    better = m < minv_ref[...]
    minv_ref[...] = jnp.where(better, m, minv_ref[...])
    mini_ref[...] = jnp.where(better, gi, mini_ref[...])

    @pl.when(k == _NOC - 1)
    def _fin():
        idx_ref[...] = mini_ref[...]


def _argmin_call(positions, obs_pos):
    return pl.pallas_call(
        _argmin_body,
        grid=(_NOC,),
        in_specs=[
            pl.BlockSpec((B, 2), lambda k: (0, 0)),
            pl.BlockSpec((_OC, 2), lambda k: (k, 0)),
        ],
        out_specs=pl.BlockSpec((1, B), lambda k: (0, 0)),
        out_shape=jax.ShapeDtypeStruct((1, B), jnp.int32),
        scratch_shapes=[
            pltpu.VMEM((1, B), jnp.float32),
            pltpu.VMEM((1, B), jnp.int32),
        ],
    )(positions, obs_pos)


_NC = 2
_NS = 16
_NW = _NC * _NS
_B_PER_W = B // _NW


@functools.lru_cache(maxsize=None)
def _make_gather_sc():
    @functools.partial(
        pl.kernel,
        mesh=plsc.VectorSubcoreMesh(core_axis_name="c", subcore_axis_name="s"),
        out_type=jax.ShapeDtypeStruct((B, N_GRAPH), jnp.float32),
        scratch_types=[
            pltpu.VMEM((_B_PER_W,), jnp.int32),
            pltpu.VMEM((_B_PER_W, N_GRAPH), jnp.float32),
            pltpu.SemaphoreType.DMA,
        ],
    )
    def _gather_sc(idx_hbm, table_hbm, out_hbm, idx_v, rows_v, sem):
        wid = lax.axis_index("s") * _NC + lax.axis_index("c")
        base = wid * _B_PER_W
        pltpu.sync_copy(idx_hbm.at[pl.ds(base, _B_PER_W)], idx_v)
        pltpu.async_copy(table_hbm.at[idx_v], rows_v, sem).wait()
        pltpu.sync_copy(rows_v, out_hbm.at[pl.ds(base, _B_PER_W)])

    return _gather_sc


_BB = 16


def _expand_body(g_ref, a_ref, s_ref, out_ref):
    inter = jnp.dot(g_ref[...], a_ref[...],
                    preferred_element_type=jnp.float32)
    for lbg in range(_BB // 4):
        cs = [[inter[lbg * 4 + q:lbg * 4 + q + 1, f:f + 1] for f in range(GF)]
              for q in range(4)]
        for it in range(OPD // 8):
            r0 = it * 8
            s_tiles = [s_ref[f * OPD + r0:f * OPD + r0 + 8, :] for f in range(GF)]
            for q in range(4):
                lb = lbg * 4 + q
                acc = cs[q][0] * s_tiles[0]
                for f in range(1, GF):
                    acc = acc + cs[q][f] * s_tiles[f]
                out_ref[lb * OPD + r0:lb * OPD + r0 + 8, :] = acc


def _expand_call(gathered, alpha, s2):
    return pl.pallas_call(
        _expand_body,
        grid=(B // _BB,),
        in_specs=[
            pl.BlockSpec((_BB, N_GRAPH), lambda i: (i, 0)),
            pl.BlockSpec((N_GRAPH, GF), lambda i: (0, 0)),
            pl.BlockSpec((GF * OPD, OPD), lambda i: (0, 0)),
        ],
        out_specs=pl.BlockSpec((_BB * OPD, OPD), lambda i: (i, 0)),
        out_shape=jax.ShapeDtypeStruct((B * OPD, OPD), jnp.float32),
    )(gathered, alpha, s2)


def kernel(positions, obs_pos, graph_dic, S_graph, alpha_graph):
    idx = _argmin_call(positions, obs_pos).reshape(B)
    gathered = _make_gather_sc()(idx, graph_dic)
    s2 = S_graph.reshape(GF * OPD, OPD)
    out = _expand_call(gathered, alpha_graph, s2).reshape(B, OPD, OPD)
    return (out, alpha_graph)


# speedup vs baseline: 2.2617x; 1.0489x over previous
import functools

import jax
import jax.numpy as jnp
from jax import lax
from jax.experimental import pallas as pl
from jax.experimental.pallas import tpu as pltpu
from jax.experimental.pallas import tpu_sc as plsc

N_OBS = 50000
N_GRAPH = 512
GF = 6
OPD = 256
B = 1024


_OC = 2000
_NOC = N_OBS // _OC


def _argmin_body(pos_ref, obs_ref, idx_ref, minv_ref, mini_ref):
    k = pl.program_id(0)
    pos_t = jnp.transpose(pos_ref[...], (1, 0))
    px = pos_t[0:1, :]
    py = pos_t[1:2, :]
    q2 = px * px + py * py

    ox = obs_ref[:, 0:1]
    oy = obs_ref[:, 1:2]
    o2 = ox * ox + oy * oy
    dot = lax.dot_general(obs_ref[...], pos_t,
                          dimension_numbers=(((1,), (0,)), ((), ())),
                          preferred_element_type=jnp.float32)
    d = (q2 + o2) - 2.0 * dot
    m = jnp.min(d, axis=0, keepdims=True)
    iota = lax.broadcasted_iota(jnp.int32, (_OC, 1), 0).astype(jnp.float32)
    la = jnp.min(jnp.where(d == m, iota, jnp.float32(_OC)),
                 axis=0, keepdims=True)
    gi = k * _OC + la.astype(jnp.int32)

    @pl.when(k == 0)
    def _init():
        minv_ref[...] = jnp.full((1, B), jnp.inf, jnp.float32)
        mini_ref[...] = jnp.zeros((1, B), jnp.int32)

    better = m < minv_ref[...]
    minv_ref[...] = jnp.where(better, m, minv_ref[...])
    mini_ref[...] = jnp.where(better, gi, mini_ref[...])

    @pl.when(k == _NOC - 1)
    def _fin():
        idx_ref[...] = mini_ref[...]


def _argmin_call(positions, obs_pos):
    return pl.pallas_call(
        _argmin_body,
        grid=(_NOC,),
        in_specs=[
            pl.BlockSpec((B, 2), lambda k: (0, 0)),
            pl.BlockSpec((_OC, 2), lambda k: (k, 0)),
        ],
        out_specs=pl.BlockSpec((1, B), lambda k: (0, 0)),
        out_shape=jax.ShapeDtypeStruct((1, B), jnp.int32),
        scratch_shapes=[
            pltpu.VMEM((1, B), jnp.float32),
            pltpu.VMEM((1, B), jnp.int32),
        ],
    )(positions, obs_pos)


_NC = 2
_NS = 16
_NW = _NC * _NS
_B_PER_W = B // _NW


@functools.lru_cache(maxsize=None)
def _make_gather_sc():
    @functools.partial(
        pl.kernel,
        mesh=plsc.VectorSubcoreMesh(core_axis_name="c", subcore_axis_name="s"),
        out_type=jax.ShapeDtypeStruct((B, N_GRAPH), jnp.float32),
        scratch_types=[
            pltpu.VMEM((_B_PER_W,), jnp.int32),
            pltpu.VMEM((_B_PER_W, N_GRAPH), jnp.float32),
            pltpu.SemaphoreType.DMA,
        ],
    )
    def _gather_sc(idx_hbm, table_hbm, out_hbm, idx_v, rows_v, sem):
        wid = lax.axis_index("s") * _NC + lax.axis_index("c")
        base = wid * _B_PER_W
        pltpu.sync_copy(idx_hbm.at[pl.ds(base, _B_PER_W)], idx_v)
        pltpu.async_copy(table_hbm.at[idx_v], rows_v, sem).wait()
        pltpu.sync_copy(rows_v, out_hbm.at[pl.ds(base, _B_PER_W)])

    return _gather_sc


_BB = 32


def _expand_body(g_ref, a_ref, s_ref, out_ref):
    inter = jnp.dot(g_ref[...], a_ref[...],
                    preferred_element_type=jnp.float32)
    for lbg in range(_BB // 4):
        cs = [[inter[lbg * 4 + q:lbg * 4 + q + 1, f:f + 1] for f in range(GF)]
              for q in range(4)]
        for it in range(OPD // 8):
            r0 = it * 8
            s_tiles = [s_ref[f * OPD + r0:f * OPD + r0 + 8, :] for f in range(GF)]
            for q in range(4):
                lb = lbg * 4 + q
                acc = cs[q][0] * s_tiles[0]
                for f in range(1, GF):
                    acc = acc + cs[q][f] * s_tiles[f]
                out_ref[lb * OPD + r0:lb * OPD + r0 + 8, :] = acc


def _expand_call(gathered, alpha, s2):
    return pl.pallas_call(
        _expand_body,
        grid=(B // _BB,),
        in_specs=[
            pl.BlockSpec((_BB, N_GRAPH), lambda i: (i, 0)),
            pl.BlockSpec((N_GRAPH, GF), lambda i: (0, 0)),
            pl.BlockSpec((GF * OPD, OPD), lambda i: (0, 0)),
        ],
        out_specs=pl.BlockSpec((_BB * OPD, OPD), lambda i: (i, 0)),
        out_shape=jax.ShapeDtypeStruct((B * OPD, OPD), jnp.float32),
    )(gathered, alpha, s2)


def kernel(positions, obs_pos, graph_dic, S_graph, alpha_graph):
    idx = _argmin_call(positions, obs_pos).reshape(B)
    gathered = _make_gather_sc()(idx, graph_dic)
    s2 = S_graph.reshape(GF * OPD, OPD)
    out = _expand_call(gathered, alpha_graph, s2).reshape(B, OPD, OPD)
    return (out, alpha_graph)


# speedup vs baseline: 2.2683x; 1.0029x over previous
import functools

import jax
import jax.numpy as jnp
from jax import lax
from jax.experimental import pallas as pl
from jax.experimental.pallas import tpu as pltpu
from jax.experimental.pallas import tpu_sc as plsc

N_OBS = 50000
N_GRAPH = 512
GF = 6
OPD = 256
B = 1024


_OC = 2000
_NOC = N_OBS // _OC


def _argmin_body(pos_ref, obs_ref, idx_ref, minv_ref, mini_ref):
    k = pl.program_id(0)
    pos_t = jnp.transpose(pos_ref[...], (1, 0))
    px = pos_t[0:1, :]
    py = pos_t[1:2, :]
    q2 = px * px + py * py

    ox = obs_ref[:, 0:1]
    oy = obs_ref[:, 1:2]
    o2 = ox * ox + oy * oy
    dot = lax.dot_general(obs_ref[...], pos_t,
                          dimension_numbers=(((1,), (0,)), ((), ())),
                          preferred_element_type=jnp.float32)
    d = (q2 + o2) - 2.0 * dot
    m = jnp.min(d, axis=0, keepdims=True)
    iota = lax.broadcasted_iota(jnp.int32, (_OC, 1), 0).astype(jnp.float32)
    la = jnp.min(jnp.where(d == m, iota, jnp.float32(_OC)),
                 axis=0, keepdims=True)
    gi = k * _OC + la.astype(jnp.int32)

    @pl.when(k == 0)
    def _init():
        minv_ref[...] = jnp.full((1, B), jnp.inf, jnp.float32)
        mini_ref[...] = jnp.zeros((1, B), jnp.int32)

    better = m < minv_ref[...]
    minv_ref[...] = jnp.where(better, m, minv_ref[...])
    mini_ref[...] = jnp.where(better, gi, mini_ref[...])

    @pl.when(k == _NOC - 1)
    def _fin():
        idx_ref[...] = mini_ref[...]


def _argmin_call(positions, obs_pos):
    return pl.pallas_call(
        _argmin_body,
        grid=(_NOC,),
        in_specs=[
            pl.BlockSpec((B, 2), lambda k: (0, 0)),
            pl.BlockSpec((_OC, 2), lambda k: (k, 0)),
        ],
        out_specs=pl.BlockSpec((1, B), lambda k: (0, 0)),
        out_shape=jax.ShapeDtypeStruct((1, B), jnp.int32),
        scratch_shapes=[
            pltpu.VMEM((1, B), jnp.float32),
            pltpu.VMEM((1, B), jnp.int32),
        ],
    )(positions, obs_pos)


_NC = 2
_NS = 16
_NW = _NC * _NS
_B_PER_W = B // _NW


@functools.lru_cache(maxsize=None)
def _make_gather_sc():
    @functools.partial(
        pl.kernel,
        mesh=plsc.VectorSubcoreMesh(core_axis_name="c", subcore_axis_name="s"),
        out_type=jax.ShapeDtypeStruct((B, N_GRAPH), jnp.float32),
        scratch_types=[
            pltpu.VMEM((_B_PER_W,), jnp.int32),
            pltpu.VMEM((_B_PER_W, N_GRAPH), jnp.float32),
            pltpu.SemaphoreType.DMA,
        ],
    )
    def _gather_sc(idx_hbm, table_hbm, out_hbm, idx_v, rows_v, sem):
        wid = lax.axis_index("s") * _NC + lax.axis_index("c")
        base = wid * _B_PER_W
        pltpu.sync_copy(idx_hbm.at[pl.ds(base, _B_PER_W)], idx_v)
        pltpu.async_copy(table_hbm.at[idx_v], rows_v, sem).wait()
        pltpu.sync_copy(rows_v, out_hbm.at[pl.ds(base, _B_PER_W)])

    return _gather_sc


_BB = 64


def _expand_body(g_ref, a_ref, s_ref, out_ref):
    inter = jnp.dot(g_ref[...], a_ref[...],
                    preferred_element_type=jnp.float32)
    for lbg in range(_BB // 4):
        cs = [[inter[lbg * 4 + q:lbg * 4 + q + 1, f:f + 1] for f in range(GF)]
              for q in range(4)]
        for it in range(OPD // 8):
            r0 = it * 8
            s_tiles = [s_ref[f * OPD + r0:f * OPD + r0 + 8, :] for f in range(GF)]
            for q in range(4):
                lb = lbg * 4 + q
                acc = cs[q][0] * s_tiles[0]
                for f in range(1, GF):
                    acc = acc + cs[q][f] * s_tiles[f]
                out_ref[lb * OPD + r0:lb * OPD + r0 + 8, :] = acc


def _expand_call(gathered, alpha, s2):
    return pl.pallas_call(
        _expand_body,
        grid=(B // _BB,),
        in_specs=[
            pl.BlockSpec((_BB, N_GRAPH), lambda i: (i, 0)),
            pl.BlockSpec((N_GRAPH, GF), lambda i: (0, 0)),
            pl.BlockSpec((GF * OPD, OPD), lambda i: (0, 0)),
        ],
        out_specs=pl.BlockSpec((_BB * OPD, OPD), lambda i: (i, 0)),
        out_shape=jax.ShapeDtypeStruct((B * OPD, OPD), jnp.float32),
    )(gathered, alpha, s2)


def kernel(positions, obs_pos, graph_dic, S_graph, alpha_graph):
    idx = _argmin_call(positions, obs_pos).reshape(B)
    gathered = _make_gather_sc()(idx, graph_dic)
    s2 = S_graph.reshape(GF * OPD, OPD)
    out = _expand_call(gathered, alpha_graph, s2).reshape(B, OPD, OPD)
    return (out, alpha_graph)
